# baseline (speedup 1.0000x reference)
import functools

import jax
import jax.numpy as jnp
from jax import lax
from jax.experimental import pallas as pl
from jax.experimental.pallas import tpu as pltpu
from jax.experimental.pallas import tpu_sc as plsc

NC = 2
NS = 16
NW = NC * NS
BC = 128
NBUF = 2


@functools.partial(jax.jit, static_argnums=(2, 3, 4))
def _emb_sc(indices, tpairs, f_n, v_n, d_n):
    b_n = indices.shape[1]
    assert b_n == BC * NW and d_n == 64

    mesh = plsc.VectorSubcoreMesh(core_axis_name="c", subcore_axis_name="s")

    @functools.partial(
        pl.kernel,
        out_type=jax.ShapeDtypeStruct((f_n, d_n // 8, b_n // BC, 8, BC),
                                      jnp.float32),
        mesh=mesh,
        compiler_params=pltpu.CompilerParams(use_tc_tiling_on_sc=True,
                                             needs_layout_passes=False),
        scratch_types=[
            pltpu.VMEM((f_n, BC), jnp.int32),
            pltpu.VMEM((f_n, BC), jnp.int32),
            pltpu.VMEM((f_n, BC), jnp.int32),
            pltpu.VMEM((NBUF, BC, 128), jnp.float32),
            pltpu.VMEM((d_n, BC), jnp.float32),
            pltpu.SemaphoreType.DMA,
            pltpu.SemaphoreType.DMA,
        ],
    )
    def sc_kernel(idx_hbm, tp_hbm, out_hbm, idx_v, pid_v, hof_v, pair_v,
                  ot_v, sem0, sem1):
        sems = [sem0, sem1]
        wid = lax.axis_index("s") * NC + lax.axis_index("c")
        b0 = wid * BC

        pltpu.sync_copy(idx_hbm.at[:, pl.ds(b0, BC)], idx_v)

        @pl.loop(0, f_n)
        def _(f):
            fbase = f * (v_n // 2)

            @pl.loop(0, BC // 16, unroll=4)
            def _(j):
                v16 = idx_v[f, pl.ds(j * 16, 16)]
                pid_v[f, pl.ds(j * 16, 16)] = fbase + (v16 >> 1)
                hof_v[f, pl.ds(j * 16, 16)] = (v16 & 1) << 6

        for b in range(NBUF):
            pltpu.async_copy(tp_hbm.at[pid_v.at[b]], pair_v.at[b], sems[b])

        @pl.loop(0, f_n, step=NBUF)
        def _(f0):
            lane = lax.iota(jnp.int32, 16)
            for b in range(NBUF):
                f = f0 + b
                pltpu.make_async_copy(
                    tp_hbm.at[pid_v.at[f]], pair_v.at[b], sems[b]).wait()

                for j in range(BC // 16):
                    row = lane + (j * 16)
                    hof = hof_v[f, pl.ds(j * 16, 16)]
                    for d in range(d_n):
                        val = plsc.load_gather(pair_v.at[b], [row, hof + d])
                        ot_v[d, pl.ds(j * 16, 16)] = val

                for dr in range(d_n // 8):
                    pltpu.sync_copy(ot_v.at[pl.ds(dr * 8, 8)],
                                    out_hbm.at[f, dr, wid])

                @pl.when(f + NBUF < f_n)
                def _():
                    pltpu.async_copy(
                        tp_hbm.at[pid_v.at[f + NBUF]], pair_v.at[b], sems[b])

    return sc_kernel(indices, tpairs)


def kernel(indices, tables):
    f, b = indices.shape
    _, v, d = tables.shape
    assert b == BC * NW and d == 64 and v % 2 == 0

    tpairs = tables.reshape(f * v // 2, 128)
    out5 = _emb_sc(indices, tpairs, f, v, d)
    return jnp.transpose(out5, (2, 4, 0, 1, 3)).reshape(b, f, d)


# speedup vs baseline: 1.0639x; 1.0639x over previous
import functools

import jax
import jax.numpy as jnp
from jax import lax
from jax.experimental import pallas as pl
from jax.experimental.pallas import tpu as pltpu
from jax.experimental.pallas import tpu_sc as plsc

NC = 2
NS = 16
NW = NC * NS
G = 128
NBUF = 2


@functools.partial(jax.jit, static_argnums=(2, 3))
def _gather_sc(g1, tables_flat, rows, d):
    ngroups = rows // G
    npw = ngroups // NW

    mesh = plsc.VectorSubcoreMesh(core_axis_name="c", subcore_axis_name="s")

    @functools.partial(
        pl.kernel,
        out_type=jax.ShapeDtypeStruct((rows, d), jnp.float32),
        mesh=mesh,
        compiler_params=pltpu.CompilerParams(use_tc_tiling_on_sc=False),
        scratch_types=[
            pltpu.VMEM((npw * G,), jnp.int32),
            pltpu.VMEM((NBUF, G, d), jnp.float32),
            pltpu.SemaphoreType.DMA,
            pltpu.SemaphoreType.DMA,
        ],
    )
    def sc_kernel(g_hbm, tab_hbm, out_hbm, idx_v, rows_v, sem0, sem1):
        sems = [sem0, sem1]
        wid = lax.axis_index("s") * NC + lax.axis_index("c")
        g0 = wid * npw

        pltpu.sync_copy(g_hbm.at[pl.ds(g0 * G, npw * G)], idx_v)

        for b in range(NBUF):
            pltpu.async_copy(
                tab_hbm.at[idx_v.at[pl.ds(b * G, G)]], rows_v.at[b], sems[b])

        @pl.loop(0, npw, step=NBUF)
        def _(j0):
            for b in range(NBUF):
                j = j0 + b
                pltpu.make_async_copy(
                    tab_hbm.at[idx_v.at[pl.ds(j * G, G)]],
                    rows_v.at[b], sems[b]).wait()
                pltpu.sync_copy(
                    rows_v.at[b], out_hbm.at[pl.ds((g0 + j) * G, G)])
                @pl.when(j + NBUF < npw)
                def _():
                    pltpu.async_copy(
                        tab_hbm.at[idx_v.at[pl.ds((j + NBUF) * G, G)]],
                        rows_v.at[b], sems[b])

    return sc_kernel(g1, tables_flat)


def kernel(indices, tables):
    f, b = indices.shape
    _, v, d = tables.shape
    rows = f * b
    assert rows % (NW * G) == 0

    tables_flat = tables.reshape(f * v, d)
    offs = (jnp.arange(f, dtype=jnp.int32) * v)[None, :]
    g1 = (indices.T + offs).reshape(rows)

    out = _gather_sc(g1, tables_flat, rows, d)
    return out.reshape(b, f, d)
